# baseline (speedup 1.0000x reference)
import functools

import jax
import jax.numpy as jnp
from jax import lax
from jax.experimental import pallas as pl
from jax.experimental.pallas import tpu as pltpu
from jax.experimental.pallas import tpu_sc as plsc

D_MODEL = 64

_NC = 2
_NS = 16
_NW = _NC * _NS

_GROUP = 128
_UNIT = 256
_GPU = _UNIT // _GROUP


def _make_kernel(n_i: int, n_j: int):
    i_per_w = n_i // _NW
    halves = i_per_w // _UNIT
    unit_bytes = _UNIT * D_MODEL * 4
    mesh = plsc.VectorSubcoreMesh(core_axis_name="c", subcore_axis_name="s")

    @functools.partial(
        pl.kernel,
        mesh=mesh,
        out_type=jax.ShapeDtypeStruct((n_j, D_MODEL, n_i), jnp.float32),
        compiler_params=pltpu.CompilerParams(
            use_tc_tiling_on_sc=False, needs_layout_passes=False
        ),
        scratch_types=(
            [pltpu.VMEM((n_j, i_per_w // _GROUP, _GROUP), jnp.int32)]
            + [pltpu.VMEM((_UNIT, D_MODEL), jnp.float32) for _ in range(2)]
            + [pltpu.VMEM((D_MODEL, _UNIT), jnp.float32) for _ in range(2)]
            + [pltpu.SemaphoreType.DMA for _ in range(4)]
        ),
    )
    def k(table_hbm, idx_hbm, out_hbm, idx_v, r0, r1, t0, t1, g0, g1, o0, o1):
        rows = (r0, r1)
        tbuf = (t0, t1)
        gsem = (g0, g1)
        osem = (o0, o1)
        wid = lax.axis_index("s") * _NC + lax.axis_index("c")
        i0 = wid * i_per_w

        pltpu.sync_copy(
            idx_hbm.at[:, pl.ds(wid * (i_per_w // _GROUP), i_per_w // _GROUP), :],
            idx_v,
        )

        iota = lax.iota(jnp.int32, 16)
        rowidx = [iota + (s * 16) for s in range(16)]

        def drain_rows(sem, buf):
            pltpu.make_async_copy(
                table_hbm.at[pl.ds(0, _UNIT), :], buf, sem
            ).wait()

        def drain_tbuf(sem, buf):
            pltpu.make_async_copy(
                out_hbm.at[0, :, pl.ds(0, _UNIT)], buf, sem
            ).wait()

        def transpose_unit(src, dst):
            def fbody(f, carry):
                col = jnp.full((16,), f, dtype=jnp.int32)
                for s in range(16):
                    v = plsc.load_gather(src, [rowidx[s], col])
                    dst[f, pl.ds(s * 16, 16)] = v
                return carry

            lax.fori_loop(0, D_MODEL, fbody, 0)

        def body(j, carry):
            for h in range(halves):
                for b in range(_GPU):
                    pltpu.async_copy(
                        table_hbm.at[idx_v.at[j, h * _GPU + b]],
                        rows[h].at[pl.ds(b * _GROUP, _GROUP), :],
                        gsem[h],
                    )
            for h in range(halves):
                drain_rows(gsem[h], rows[h])

                @pl.when(j > 0)
                def _():
                    drain_tbuf(osem[h], tbuf[h])

                transpose_unit(rows[h], tbuf[h])
                pltpu.async_copy(
                    tbuf[h],
                    out_hbm.at[j, :, pl.ds(i0 + h * _UNIT, _UNIT)],
                    osem[h],
                )
            return carry

        lax.fori_loop(0, n_j, body, 0)
        for h in range(halves):
            drain_tbuf(osem[h], tbuf[h])

    return k


def kernel(x, table):
    n_i, n_j = x.shape
    xt = jnp.reshape(jnp.transpose(x), (n_j, n_i // _GROUP, _GROUP)).astype(
        jnp.int32
    )
    out = _make_kernel(n_i, n_j)(table, xt)
    return jnp.transpose(out, (2, 0, 1))


# speedup vs baseline: 1.3811x; 1.3811x over previous
import functools

import jax
import jax.numpy as jnp
from jax import lax
from jax.experimental import pallas as pl
from jax.experimental.pallas import tpu as pltpu
from jax.experimental.pallas import tpu_sc as plsc

D_MODEL = 64

_NC = 2
_NS = 16
_NW = _NC * _NS

_GROUP = 128
_NBUF = 5

_TW = 8192


def _transpose_kernel(n_vocab: int):
    steps = (n_vocab + _TW - 1) // _TW

    def body(tt_ref, out_ref):
        x = tt_ref[...]
        xs = jnp.concatenate([x[:, 0::2], x[:, 1::2]], axis=0)
        y = jnp.transpose(xs, (1, 0))
        out_ref[...] = jnp.reshape(y, (_TW * D_MODEL,))

    return pl.pallas_call(
        body,
        grid=(steps,),
        in_specs=[
            pl.BlockSpec((D_MODEL, _TW), lambda c: (0, c)),
        ],
        out_specs=pl.BlockSpec((_TW * D_MODEL,), lambda c: (c,)),
        out_shape=jax.ShapeDtypeStruct((n_vocab * D_MODEL,), jnp.float32),
        compiler_params=pltpu.CompilerParams(
            dimension_semantics=("arbitrary",)
        ),
    )


def _gather_kernel(n_rows: int):
    rows_per_w = n_rows // _NW
    groups = rows_per_w // _GROUP
    mesh = plsc.VectorSubcoreMesh(core_axis_name="c", subcore_axis_name="s")

    @functools.partial(
        pl.kernel,
        mesh=mesh,
        out_type=jax.ShapeDtypeStruct((n_rows, D_MODEL), jnp.float32),
        compiler_params=pltpu.CompilerParams(use_tc_tiling_on_sc=False),
        scratch_types=(
            [pltpu.VMEM((groups, _GROUP), jnp.int32)]
            + [pltpu.VMEM((_GROUP, D_MODEL), jnp.float32) for _ in range(_NBUF)]
            + [pltpu.SemaphoreType.DMA for _ in range(2 * _NBUF)]
        ),
    )
    def k(table_hbm, idx_hbm, out_hbm, idx_v, *rest):
        rows = rest[:_NBUF]
        gsem = rest[_NBUF : 2 * _NBUF]
        osem = rest[2 * _NBUF :]
        wid = lax.axis_index("s") * _NC + lax.axis_index("c")
        base = wid * rows_per_w

        pltpu.sync_copy(idx_hbm.at[wid], idx_v)

        def fire(g, b):
            pltpu.async_copy(table_hbm.at[idx_v.at[g]], rows[b], gsem[b])

        def drain_g(b):
            pltpu.make_async_copy(
                table_hbm.at[pl.ds(0, _GROUP), :], rows[b], gsem[b]
            ).wait()

        def drain_o(b):
            pltpu.make_async_copy(
                table_hbm.at[pl.ds(0, _GROUP), :], rows[b], osem[b]
            ).wait()

        for b in range(_NBUF):
            fire(b, b)

        def body(s, carry):
            for b in range(_NBUF):
                g = _NBUF * s + b
                drain_g(b)
                pltpu.async_copy(
                    rows[b],
                    out_hbm.at[pl.ds(base + g * _GROUP, _GROUP), :],
                    osem[b],
                )
            for b in range(_NBUF):
                gn = _NBUF * (s + 1) + b

                @pl.when(gn < groups)
                def _():
                    drain_o(b)
                    fire(gn, b)

            return carry

        lax.fori_loop(0, groups // _NBUF, body, 0)
        for b in range(_NBUF):
            drain_o(b)

    return k


def kernel(x, table):
    n_i, n_j = x.shape
    n_rows = n_i * n_j
    n_vocab = table.shape[0]
    idx = x.reshape(_NW, (n_rows // _NW) // _GROUP, _GROUP).astype(jnp.int32)
    del n_vocab
    out = _gather_kernel(n_rows)(table, idx)
    return out.reshape(n_i, n_j, D_MODEL)
